# baseline (speedup 1.0000x reference)
import jax
import jax.numpy as jnp
from jax.experimental import pallas as pl
from jax.experimental.pallas import tpu as pltpu

B, M, N = 4, 4096, 2048
RC = 256
LB = 512


def _tc_body(x_ref, o_ref, carry):
    r = pl.program_id(2)

    @pl.when(r == 0)
    def _():
        carry[...] = jnp.zeros_like(carry)

    ri = jax.lax.broadcasted_iota(jnp.int32, (RC, RC), 0)
    ci = jax.lax.broadcasted_iota(jnp.int32, (RC, RC), 1)
    tri = (ri >= ci).astype(jnp.float32)
    s = jnp.dot(tri, x_ref[0], preferred_element_type=jnp.float32) + carry[...]
    o_ref[0] = s
    carry[...] = jax.lax.slice_in_dim(s, RC - 1, RC, axis=0)


@jax.jit
def kernel(x):
    grid = (B, N // LB, M // RC)
    return pl.pallas_call(
        _tc_body,
        grid=grid,
        in_specs=[
            pl.BlockSpec((1, RC, LB), lambda b, l, r: (b, r, l)),
        ],
        out_specs=pl.BlockSpec((1, RC, LB), lambda b, l, r: (b, r, l)),
        out_shape=jax.ShapeDtypeStruct((B, M, N), jnp.float32),
        scratch_shapes=[pltpu.VMEM((1, LB), jnp.float32)],
        compiler_params=pltpu.CompilerParams(
            dimension_semantics=("parallel", "parallel", "arbitrary"),
        ),
    )(x)


# speedup vs baseline: 1.8200x; 1.8200x over previous
import functools

import jax
import jax.numpy as jnp
from jax import lax
from jax.experimental import pallas as pl
from jax.experimental.pallas import tpu as pltpu
from jax.experimental.pallas import tpu_sc as plsc

B, M, N = 4, 4096, 2048

NUM_CORES = 2
NUM_SUBCORES = 16
LANES = 16
NUM_WORKERS = NUM_CORES * NUM_SUBCORES

STRIP_LANES = 256
NUM_STRIPS = B * (N // STRIP_LANES)
STRIPS_PER_WORKER = NUM_STRIPS // NUM_WORKERS
NVEC = STRIP_LANES // LANES
CHUNK = 128
NUM_CHUNKS = M // CHUNK
BLOCKS_PER_ROW = N // STRIP_LANES
TOTAL_ITERS = STRIPS_PER_WORKER * NUM_CHUNKS


def _hbm_slice(ref, it, wid):
    strip, ch = divmod(it, NUM_CHUNKS)
    sid = wid + NUM_WORKERS * strip
    b = sid // BLOCKS_PER_ROW
    lane0 = pl.multiple_of((sid % BLOCKS_PER_ROW) * STRIP_LANES, STRIP_LANES)
    return ref.at[b, pl.ds(ch * CHUNK, CHUNK), pl.ds(lane0, STRIP_LANES)]


NBUF = 3
AHEAD = 2


def _body(x_hbm, out_hbm, *refs):
    bufs = refs[:NBUF]
    sems_in = refs[NBUF : 2 * NBUF]
    sems_out = refs[2 * NBUF : 3 * NBUF]
    core = lax.axis_index("c")
    sub = lax.axis_index("s")
    wid = sub * NUM_CORES + core

    in_copies = [None] * NBUF
    out_copies = [None] * NBUF

    for it in range(min(AHEAD, TOTAL_ITERS)):
        in_copies[it % NBUF] = pltpu.async_copy(
            _hbm_slice(x_hbm, it, wid), bufs[it % NBUF], sems_in[it % NBUF]
        )

    carries = None
    for it in range(TOTAL_ITERS):
        nb = it % NBUF

        nxt = it + AHEAD
        if nxt < TOTAL_ITERS and in_copies[nxt % NBUF] is None:
            s = nxt % NBUF
            if out_copies[s] is not None:
                out_copies[s].wait()
                out_copies[s] = None
            in_copies[s] = pltpu.async_copy(
                _hbm_slice(x_hbm, nxt, wid), bufs[s], sems_in[s]
            )

        if it % NUM_CHUNKS == 0:
            carries = tuple(jnp.zeros((LANES,), jnp.float32) for _ in range(NVEC))

        in_copies[nb].wait()
        in_copies[nb] = None
        buf = bufs[nb]

        def row(r, carry):
            new = []
            for j in range(NVEC):
                v = carry[j] + buf[r, pl.ds(j * LANES, LANES)]
                buf[r, pl.ds(j * LANES, LANES)] = v
                new.append(v)
            return tuple(new)

        carries = lax.fori_loop(0, CHUNK, row, carries)

        out_copies[nb] = pltpu.async_copy(
            buf, _hbm_slice(out_hbm, it, wid), sems_out[nb]
        )

    for nb in range(NBUF):
        if out_copies[nb] is not None:
            out_copies[nb].wait()


@jax.jit
def kernel(x):
    mesh = plsc.VectorSubcoreMesh(
        core_axis_name="c", subcore_axis_name="s"
    )
    run = functools.partial(
        pl.kernel,
        out_type=jax.ShapeDtypeStruct((B, M, N), jnp.float32),
        mesh=mesh,
        scratch_types=(
            [pltpu.VMEM((CHUNK, STRIP_LANES), jnp.float32)] * NBUF
            + [pltpu.SemaphoreType.DMA] * (2 * NBUF)
        ),
    )(_body)
    return run(x)
